# Initial kernel scaffold; baseline (speedup 1.0000x reference)
import jax
import jax.numpy as jnp
from jax.experimental import pallas as pl


def kernel(q_tables, pos, target_val, lr, act):
    raise NotImplementedError("write your pallas kernel here")


# speedup vs baseline: 3.0202x; 3.0202x over previous
import functools

import jax
import jax.numpy as jnp
from jax import lax
from jax.experimental import pallas as pl
from jax.experimental.pallas import tpu as pltpu
from jax.experimental.pallas import tpu_sc as plsc

_N_STATES = 1000 * 1000
_N_ACTIONS = 16
_BATCH = 16384
_FLAT = _N_STATES * _N_ACTIONS

_NC = 2
_NS = 16
_NW = _NC * _NS
_CHUNK = _BATCH // _NW
_L = 16
_IDXW = 128
_G = _CHUNK // _IDXW

_ROWS2D = _FLAT // 128
_CBLK = 1000


def _copy_body(x_ref, o_ref):
    o_ref[...] = x_ref[...]


_tc_copy = pl.pallas_call(
    _copy_body,
    out_shape=jax.ShapeDtypeStruct((_ROWS2D, 128), jnp.float32),
    grid=(_ROWS2D // _CBLK,),
    in_specs=[pl.BlockSpec((_CBLK, 128), lambda i: (i, 0))],
    out_specs=pl.BlockSpec((_CBLK, 128), lambda i: (i, 0)),
)


_sc_mesh = plsc.VectorSubcoreMesh(core_axis_name="c", subcore_axis_name="s")


@functools.partial(
    pl.kernel,
    mesh=_sc_mesh,
    scratch_types=[
        pltpu.VMEM((_CHUNK,), jnp.int32),
        pltpu.VMEM((_CHUNK,), jnp.int32),
        pltpu.VMEM((_CHUNK,), jnp.float32),
        pltpu.VMEM((_G, _IDXW), jnp.int32),
        pltpu.VMEM((_G, _IDXW), jnp.float32),
        pltpu.VMEM((_L,), jnp.float32),
        pltpu.SemaphoreType.DMA,
    ],
)
def _sc_update(q_ref, pos_hbm, act_hbm, tgt_hbm, lr_hbm,
               pos_v, act_v, tgt_v, idx_v, val_v, lr_v, sem):
    wid = lax.axis_index("s") * _NC + lax.axis_index("c")
    base = wid * _CHUNK
    pltpu.sync_copy(pos_hbm.at[pl.ds(base, _CHUNK)], pos_v)
    pltpu.sync_copy(act_hbm.at[pl.ds(base, _CHUNK)], act_v)
    pltpu.sync_copy(tgt_hbm.at[pl.ds(base, _CHUNK)], tgt_v)
    pltpu.sync_copy(lr_hbm, lr_v)

    for g in range(_G):
        for i in range(_IDXW // _L):
            s = pl.ds(g * _IDXW + i * _L, _L)
            idx_v[g, pl.ds(i * _L, _L)] = pos_v[s] * _N_ACTIONS + act_v[s]

    for g in range(_G):
        pltpu.async_copy(q_ref.at[idx_v.at[g]], val_v.at[g], sem).wait()

    lr_b = lr_v[...]
    for g in range(_G):
        for i in range(_IDXW // _L):
            cur = val_v[g, pl.ds(i * _L, _L)]
            tgt = tgt_v[pl.ds(g * _IDXW + i * _L, _L)]
            val_v[g, pl.ds(i * _L, _L)] = cur + lr_b * (tgt - cur)

    for g in range(_G):
        pltpu.async_copy(val_v.at[g], q_ref.at[idx_v.at[g]], sem).wait()


def kernel(q_tables, pos, target_val, lr, act):
    q2 = q_tables.reshape(_ROWS2D, 128)
    out_flat = _tc_copy(q2).reshape(_FLAT)
    q_ref = jax.new_ref(out_flat)
    lr16 = jnp.broadcast_to(lr, (_L,))
    _sc_update(q_ref, pos, act, target_val, lr16)
    return q_ref[...].reshape(_N_STATES, _N_ACTIONS)


# speedup vs baseline: 3.3215x; 1.0998x over previous
import functools

import jax
import jax.numpy as jnp
from jax import lax
from jax.experimental import pallas as pl
from jax.experimental.pallas import tpu as pltpu
from jax.experimental.pallas import tpu_sc as plsc

_N_STATES = 1000 * 1000
_N_ACTIONS = 16
_BATCH = 16384
_FLAT = _N_STATES * _N_ACTIONS

_NC = 2
_NS = 16
_NW = _NC * _NS
_CHUNK = _BATCH // _NW
_L = 16
_IDXW = 128
_G = _CHUNK // _IDXW

_sc_mesh = plsc.VectorSubcoreMesh(core_axis_name="c", subcore_axis_name="s")


@functools.partial(
    pl.kernel,
    mesh=_sc_mesh,
    scratch_types=[
        pltpu.VMEM((_CHUNK,), jnp.int32),
        pltpu.VMEM((_CHUNK,), jnp.int32),
        pltpu.VMEM((_CHUNK,), jnp.float32),
        pltpu.VMEM((_G, _IDXW), jnp.int32),
        pltpu.VMEM((_G, _IDXW), jnp.float32),
        pltpu.VMEM((_L,), jnp.float32),
        pltpu.SemaphoreType.DMA,
    ],
)
def _sc_update(q_ref, pos_hbm, act_hbm, tgt_hbm, lr_hbm,
               pos_v, act_v, tgt_v, idx_v, val_v, lr_v, sem):
    wid = lax.axis_index("s") * _NC + lax.axis_index("c")
    base = wid * _CHUNK
    pltpu.sync_copy(pos_hbm.at[pl.ds(base, _CHUNK)], pos_v)
    pltpu.sync_copy(act_hbm.at[pl.ds(base, _CHUNK)], act_v)
    pltpu.sync_copy(tgt_hbm.at[pl.ds(base, _CHUNK)], tgt_v)
    pltpu.sync_copy(lr_hbm, lr_v)

    for g in range(_G):
        for i in range(_IDXW // _L):
            s = pl.ds(g * _IDXW + i * _L, _L)
            idx_v[g, pl.ds(i * _L, _L)] = pos_v[s] * _N_ACTIONS + act_v[s]

    gathers = [
        pltpu.async_copy(q_ref.at[idx_v.at[g]], val_v.at[g], sem)
        for g in range(_G)
    ]
    lr_b = lr_v[...]
    for g in range(_G):
        gathers[g].wait()
        for i in range(_IDXW // _L):
            cur = val_v[g, pl.ds(i * _L, _L)]
            tgt = tgt_v[pl.ds(g * _IDXW + i * _L, _L)]
            val_v[g, pl.ds(i * _L, _L)] = cur + lr_b * (tgt - cur)

    scatters = [
        pltpu.async_copy(val_v.at[g], q_ref.at[idx_v.at[g]], sem)
        for g in range(_G)
    ]
    for c in scatters:
        c.wait()


def kernel(q_tables, pos, target_val, lr, act):
    out_flat = q_tables.reshape(_FLAT)
    q_ref = jax.new_ref(out_flat)
    lr16 = jnp.broadcast_to(lr, (_L,))
    _sc_update(q_ref, pos, act, target_val, lr16)
    return q_ref[...].reshape(_N_STATES, _N_ACTIONS)


# speedup vs baseline: 34.9741x; 10.5297x over previous
import functools

import jax
import jax.numpy as jnp
from jax import lax
from jax.experimental import pallas as pl
from jax.experimental.pallas import tpu as pltpu
from jax.experimental.pallas import tpu_sc as plsc

_N_STATES = 1000 * 1000
_N_ACTIONS = 16
_BATCH = 16384

_NC = 2
_NS = 16
_NW = _NC * _NS
_L = 16

_NB = _N_STATES // 128
_TAIL0 = _NB * 128
_W = 1920
_NSUB = 17
_NVEC = _BATCH // _L

_sc_mesh = plsc.VectorSubcoreMesh(core_axis_name="c", subcore_axis_name="s")


@functools.partial(
    pl.kernel,
    mesh=_sc_mesh,
    out_type=jax.ShapeDtypeStruct((_N_ACTIONS, _N_STATES), jnp.float32),
    compiler_params=pltpu.CompilerParams(needs_layout_passes=False),
    scratch_types=[
        pltpu.VMEM((_BATCH,), jnp.int32),
        pltpu.VMEM((_BATCH,), jnp.int32),
        pltpu.VMEM((_BATCH,), jnp.float32),
        pltpu.VMEM((_BATCH + _L,), jnp.int32),
        pltpu.VMEM((_N_ACTIONS, _W), jnp.float32),
        pltpu.VMEM((_N_ACTIONS, _W), jnp.float32),
        pltpu.VMEM((_L,), jnp.float32),
        pltpu.SemaphoreType.DMA,
        pltpu.SemaphoreType.DMA,
        pltpu.SemaphoreType.DMA,
        pltpu.SemaphoreType.DMA,
    ],
)
def _sc_copy_update(qT_hbm, pos_hbm, act_hbm, tgt_hbm, lr_hbm, outT_hbm,
                    pos_v, act_v, tgt_v, cid_v, buf_a, buf_b, lr_v,
                    sem_ia, sem_ib, sem_oa, sem_ob):
    wid = lax.axis_index("s") * _NC + lax.axis_index("c")
    b_lane = (wid * _NB) // _NW * 128
    e_lane = ((wid + 1) * _NB) // _NW * 128

    pltpu.sync_copy(pos_hbm, pos_v)
    pltpu.sync_copy(act_hbm, act_v)
    pltpu.sync_copy(tgt_hbm, tgt_v)
    pltpu.sync_copy(lr_hbm, lr_v)
    lr_b = lr_v[...]
    iota = lax.iota(jnp.int32, _L)

    @pl.loop(0, _NVEC, init_carry=jnp.int32(0))
    def _scan(i, cnt):
        p = pos_v[pl.ds(i * _L, _L)]
        m = (p >= b_lane) & (p < e_lane)
        plsc.store_compressed(cid_v.at[pl.ds(cnt, _L)], iota + i * _L, mask=m)
        return cnt + jnp.max(plsc.all_reduce_population_count(m))

    n_upd = _scan

    def _apply(buf, s_lane):
        nv = (n_upd + _L - 1) // _L

        @pl.loop(0, nv)
        def _inner(v):
            lanes = iota + v * _L
            lane_ok = lanes < n_upd
            e = cid_v[pl.ds(v * _L, _L)]
            e = jnp.where(lane_ok, e, 0)
            p = plsc.load_gather(pos_v, [e])
            win = lane_ok & (p >= s_lane) & (p < s_lane + _W)
            a = plsc.load_gather(act_v, [e])
            t = plsc.load_gather(tgt_v, [e])
            sloc = jnp.where(win, p - s_lane, 0)
            a = jnp.where(win, a, 0)
            cur = plsc.load_gather(buf, [a, sloc], mask=win)
            new = cur + lr_b * (t - cur)
            plsc.store_scatter(buf, [a, sloc], new, mask=win)

    def _start(j):
        s_j = jnp.minimum(b_lane + j * _W, e_lane - _W)
        buf = buf_a if j % 2 == 0 else buf_b
        sem = sem_ia if j % 2 == 0 else sem_ib
        cp = pltpu.async_copy(qT_hbm.at[:, pl.ds(s_j, _W)], buf, sem)
        return s_j, buf, cp

    outs = [None, None]
    nxt = _start(0)
    for j in range(_NSUB):
        s_j, buf, cp_in = nxt
        cp_in.wait()
        if outs[j % 2] is not None:
            outs[j % 2].wait()
            outs[j % 2] = None
        if j + 1 < _NSUB:
            nxt = _start(j + 1)
        _apply(buf, s_j)
        sem_o = sem_oa if j % 2 == 0 else sem_ob
        outs[j % 2] = pltpu.async_copy(buf, outT_hbm.at[:, pl.ds(s_j, _W)],
                                       sem_o)
    for o in outs:
        if o is not None:
            o.wait()


def kernel(q_tables, pos, target_val, lr, act):
    lr16 = jnp.broadcast_to(lr, (_L,))
    outT = _sc_copy_update(q_tables.T, pos, act, target_val, lr16)

    tail_cur = lax.slice(q_tables, (_TAIL0, 0), (_N_STATES, _N_ACTIONS))
    keys = pos * _N_ACTIONS + act
    tail_keys = (_TAIL0 + jnp.arange(64)[:, None]) * _N_ACTIONS \
        + jnp.arange(_N_ACTIONS)[None, :]
    hit = tail_keys[:, :, None] == keys[None, None, :]
    rank = jnp.where(hit, jnp.arange(_BATCH, dtype=jnp.int32) + 1, 0)
    last = jnp.max(rank, axis=-1)
    t_sel = target_val[jnp.maximum(last - 1, 0)]
    upd = tail_cur + lr[0] * (t_sel - tail_cur)
    tail_new = jnp.where(last > 0, upd, tail_cur)
    outT = lax.dynamic_update_slice(outT, tail_new.T, (0, _TAIL0))
    return outT.T


# speedup vs baseline: 36.9983x; 1.0579x over previous
import functools

import jax
import jax.numpy as jnp
from jax import lax
from jax.experimental import pallas as pl
from jax.experimental.pallas import tpu as pltpu
from jax.experimental.pallas import tpu_sc as plsc

_N_STATES = 1000 * 1000
_N_ACTIONS = 16
_BATCH = 16384

_NC = 2
_NS = 16
_NW = _NC * _NS
_L = 16

_NB = _N_STATES // 128
_TAILB = (_NB - 1) * 128
_W = 1792
_WT = 256
_WTV = _N_STATES - _TAILB
_NSUB = 18
_NVEC = _BATCH // _L

_sc_mesh = plsc.VectorSubcoreMesh(core_axis_name="c", subcore_axis_name="s")


@functools.partial(
    pl.kernel,
    mesh=_sc_mesh,
    out_type=(jax.ShapeDtypeStruct((_N_ACTIONS, _N_STATES), jnp.float32),
              jax.ShapeDtypeStruct((_N_ACTIONS, _WT), jnp.float32)),
    compiler_params=pltpu.CompilerParams(needs_layout_passes=False),
    scratch_types=[
        pltpu.VMEM((_BATCH,), jnp.int32),
        pltpu.VMEM((_BATCH,), jnp.int32),
        pltpu.VMEM((_BATCH,), jnp.float32),
        pltpu.VMEM((_BATCH + _L,), jnp.int32),
        pltpu.VMEM((_N_ACTIONS, _W), jnp.float32),
        pltpu.VMEM((_N_ACTIONS, _W), jnp.float32),
        pltpu.VMEM((_N_ACTIONS, _WT), jnp.float32),
        pltpu.VMEM((_L,), jnp.float32),
        pltpu.SemaphoreType.DMA,
        pltpu.SemaphoreType.DMA,
        pltpu.SemaphoreType.DMA,
        pltpu.SemaphoreType.DMA,
        pltpu.SemaphoreType.DMA,
    ],
)
def _sc_copy_update(qT_hbm, tail_hbm, pos_hbm, act_hbm, tgt_hbm, lr_hbm,
                    outT_hbm, otail_hbm,
                    pos_v, act_v, tgt_v, cid_v, buf_a, buf_b, buf_t, lr_v,
                    sem_ia, sem_ib, sem_oa, sem_ob, sem_x):
    wid = lax.axis_index("s") * _NC + lax.axis_index("c")
    is_last = wid == _NW - 1
    b_lane = (wid * _NB) // _NW * 128
    e_lane = ((wid + 1) * _NB) // _NW * 128

    def _start_in(j):
        s_j = jnp.minimum(b_lane + j * _W, e_lane - _W)
        buf = buf_a if j % 2 == 0 else buf_b
        sem = sem_ia if j % 2 == 0 else sem_ib
        cp = pltpu.async_copy(qT_hbm.at[:, pl.ds(s_j, _W)], buf, sem)
        return s_j, buf, cp

    pltpu.sync_copy(pos_hbm, pos_v)
    ins = [_start_in(0), _start_in(1)]
    cp_act = pltpu.async_copy(act_hbm, act_v, sem_x)
    cp_tgt = pltpu.async_copy(tgt_hbm, tgt_v, sem_x)
    cp_lr = pltpu.async_copy(lr_hbm, lr_v, sem_x)
    iota = lax.iota(jnp.int32, _L)

    e_scan = jnp.where(is_last, _N_STATES, e_lane)

    @pl.loop(0, _NVEC, init_carry=jnp.int32(0), unroll=4)
    def _scan(i, cnt):
        p = pos_v[pl.ds(i * _L, _L)]
        m = (p >= b_lane) & (p < e_scan)
        plsc.store_compressed(cid_v.at[pl.ds(cnt, _L)], iota + i * _L, mask=m)
        return cnt + jnp.max(plsc.all_reduce_population_count(m))

    n_upd = _scan
    cp_act.wait()
    cp_tgt.wait()
    cp_lr.wait()
    lr_b = lr_v[...]

    def _apply(buf, s_lane, width):
        nv = (n_upd + _L - 1) // _L

        @pl.loop(0, nv)
        def _inner(v):
            lanes = iota + v * _L
            lane_ok = lanes < n_upd
            e = cid_v[pl.ds(v * _L, _L)]
            e = jnp.where(lane_ok, e, 0)
            p = plsc.load_gather(pos_v, [e])
            win = lane_ok & (p >= s_lane) & (p < s_lane + width)
            a = plsc.load_gather(act_v, [e])
            t = plsc.load_gather(tgt_v, [e])
            sloc = jnp.where(win, p - s_lane, 0)
            a = jnp.where(win, a, 0)
            cur = plsc.load_gather(buf, [a, sloc], mask=win)
            new = cur + lr_b * (t - cur)
            plsc.store_scatter(buf, [a, sloc], new, mask=win)

    outs = [None, None]
    for j in range(_NSUB):
        s_j, buf, cp_in = ins[j % 2]
        cp_in.wait()
        _apply(buf, s_j, _W)
        sem_o = sem_oa if j % 2 == 0 else sem_ob
        outs[j % 2] = pltpu.async_copy(buf, outT_hbm.at[:, pl.ds(s_j, _W)],
                                       sem_o)
        if j + 2 < _NSUB:
            outs[j % 2].wait()
            outs[j % 2] = None
            ins[j % 2] = _start_in(j + 2)

    @pl.when(is_last)
    def _tail():
        pltpu.async_copy(tail_hbm, buf_t, sem_x).wait()
        _apply(buf_t, _TAILB, _WTV)
        pltpu.async_copy(buf_t, otail_hbm, sem_x).wait()

    for o in outs:
        if o is not None:
            o.wait()


def kernel(q_tables, pos, target_val, lr, act):
    lr16 = jnp.broadcast_to(lr, (_L,))
    tail_in = lax.slice(q_tables.T, (0, _TAILB), (_N_ACTIONS, _N_STATES))
    tail_in = jnp.pad(tail_in, ((0, 0), (0, _WT - _WTV)))
    outT, otail = _sc_copy_update(q_tables.T, tail_in, pos, act, target_val,
                                  lr16)
    otail = lax.slice(otail, (0, 0), (_N_ACTIONS, _WTV))
    outT = lax.dynamic_update_slice(outT, otail, (0, _TAILB))
    return outT.T
